# baseline (speedup 1.0000x reference)
import jax
import jax.numpy as jnp
import numpy as np
from jax.experimental import pallas as pl
from jax.experimental.pallas import tpu as pltpu


def _gate_permutation(out_features):
    O = 4 * out_features
    G = O // 3
    g4 = out_features // 3
    return np.asarray([clr * G + j * g4 + t
                       for j in range(4) for clr in range(3) for t in range(g4)])


def _rows_per_block(H, max_rows):
    for r in range(min(H, max_rows), 0, -1):
        if H % r == 0:
            return r
    return 1


def _make_prep_body(C, F, Fp, Op):
    O = 4 * F
    G = O // 3
    g4 = F // 3

    def permcols(a):
        R = a.shape[0]
        pieces = []
        for j in range(4):
            for clr in range(3):
                pieces.append(a[:, clr * G + j * g4: clr * G + (j + 1) * g4])
            pieces.append(jnp.zeros((R, Fp - F), a.dtype))
        return jnp.concatenate(pieces, axis=1)

    def body(wi_ref, ws_ref, b1_ref, b2_ref, owi_ref, ows_ref, ob_ref):
        lane = jax.lax.broadcasted_iota(jnp.int32, (1, Op), 1)
        sc = jnp.where(lane < 3 * Fp, 0.5, 1.0)
        owi_ref[...] = (permcols(wi_ref[...]) * sc).astype(jnp.bfloat16)
        pw = (permcols(ws_ref[...]) * sc).astype(jnp.bfloat16)
        ows_ref[...] = jnp.zeros_like(ows_ref)
        for k in range(3):
            ows_ref[k * Fp:k * Fp + F, :] = pw[k * F:(k + 1) * F, :]
        ob_ref[...] = permcols(b1_ref[...] + b2_ref[...]) * sc

    return body


def _make_body(ROWS, Bt, W, C, F, Fp, Op):
    M = Bt * W

    def body(xt_ref, wi_ref, ws_ref, b_ref, out_ref, i2s_ref, h_ref, c_ref):
        @pl.when(pl.program_id(1) == 0)
        def _init():
            h_ref[...] = jnp.zeros_like(h_ref)
            c_ref[...] = jnp.zeros_like(c_ref)

        xblk = xt_ref[...]
        xleft = jnp.concatenate(
            [jnp.zeros((ROWS, Bt, 1, C), xblk.dtype), xblk[:, :, :W - 1, :]],
            axis=2)
        xcat = jnp.concatenate([xleft, xblk], axis=3).reshape(ROWS * M, 2 * C)
        i2s_ref[...] = (
            jnp.dot(xcat, wi_ref[...], preferred_element_type=jnp.float32)
            + b_ref[...]).reshape(ROWS, M, Op)

        ws = ws_ref[...]

        NS = 4
        Mh = M // NS
        wpos = jax.lax.broadcasted_iota(jnp.int32, (Mh, 1), 0) % W
        not_first = wpos != 0
        not_last = wpos != (W - 1)

        def half_step(r, s):
            sl = pl.ds(s * Mh, Mh)
            h = h_ref[sl, :]
            zero = jnp.zeros_like(h)
            h_l = jnp.where(not_first, pltpu.roll(h, 1, 0), zero)
            h_r = jnp.where(not_last, pltpu.roll(h, Mh - 1, 0), zero)
            hcat = jnp.concatenate([h_l, h, h_r], axis=1)
            gates = i2s_ref[r, sl, :] + jnp.dot(
                hcat, ws, preferred_element_type=jnp.float32)
            t = jnp.tanh(gates)
            f_g = 0.5 * t[:, 0 * Fp:1 * Fp] + 0.5
            i_g = 0.5 * t[:, 1 * Fp:2 * Fp] + 0.5
            o_g = 0.5 * t[:, 2 * Fp:3 * Fp] + 0.5
            g_g = t[:, 3 * Fp:4 * Fp]
            c_new = f_g * c_ref[sl, :] + i_g * g_g
            c_ref[sl, :] = c_new
            h_new = o_g * jnp.tanh(c_new)
            hb = h_new.astype(jnp.bfloat16)
            h_ref[sl, :] = hb
            out_ref[r, sl, :] = hb[:, :F]

        for r in range(ROWS):
            for s in range(NS):
                half_step(r, s)

    return body


def kernel(x, w_i2s_masked, b_i2s, w_s2s, b_s2s):
    B, C, H, W = x.shape
    F = w_s2s.shape[1]
    O = 4 * F
    Fp = ((F + 127) // 128) * 128
    Op = 4 * Fp
    BT = 1
    Bt = B // BT
    ROWS = _rows_per_block(H, 12)

    wi3t = jnp.transpose(w_i2s_masked[:, :, 0, 0:2], (2, 1, 0)).reshape(2 * C, O)
    ws3t = jnp.transpose(w_s2s, (2, 1, 0)).reshape(3 * F, O)
    wi, ws, bias = pl.pallas_call(
        _make_prep_body(C, F, Fp, Op),
        out_shape=(jax.ShapeDtypeStruct((2 * C, Op), jnp.bfloat16),
                   jax.ShapeDtypeStruct((3 * Fp, Op), jnp.bfloat16),
                   jax.ShapeDtypeStruct((1, Op), jnp.float32)),
    )(wi3t, ws3t, b_i2s.reshape(1, O), b_s2s.reshape(1, O))

    xt = jnp.transpose(x, (2, 0, 3, 1)).astype(jnp.bfloat16)

    grid = (BT, H // ROWS)
    body = _make_body(ROWS, Bt, W, C, F, Fp, Op)

    out = pl.pallas_call(
        body,
        out_shape=jax.ShapeDtypeStruct((H, B * W, F), jnp.bfloat16),
        grid_spec=pltpu.PrefetchScalarGridSpec(
            num_scalar_prefetch=0,
            grid=grid,
            in_specs=[
                pl.BlockSpec((ROWS, Bt, W, C), lambda bt, rb: (rb, bt, 0, 0)),
                pl.BlockSpec((2 * C, Op), lambda bt, rb: (0, 0)),
                pl.BlockSpec((3 * Fp, Op), lambda bt, rb: (0, 0)),
                pl.BlockSpec((1, Op), lambda bt, rb: (0, 0)),
            ],
            out_specs=pl.BlockSpec((ROWS, Bt * W, F), lambda bt, rb: (rb, bt, 0)),
            scratch_shapes=[
                pltpu.VMEM((ROWS, Bt * W, Op), jnp.float32),
                pltpu.VMEM((Bt * W, Fp), jnp.bfloat16),
                pltpu.VMEM((Bt * W, Fp), jnp.float32),
            ],
        ),
        compiler_params=pltpu.CompilerParams(
            dimension_semantics=("parallel", "arbitrary")),
    )(xt, wi, ws, bias)

    return jnp.transpose(out.reshape(H, B, W, F),
                         (1, 3, 0, 2)).astype(jnp.float32)


# speedup vs baseline: 1.1027x; 1.1027x over previous
import jax
import jax.numpy as jnp
from jax.experimental import pallas as pl
from jax.experimental.pallas import tpu as pltpu


def _rows_per_block(H, max_rows):
    for r in range(min(H, max_rows), 0, -1):
        if H % r == 0:
            return r
    return 1


def _make_prep_body(C, F, Fp, Op):
    O = 4 * F
    G = O // 3
    g4 = F // 3

    def permcols(a):
        R = a.shape[0]
        pieces = []
        for j in range(4):
            for clr in range(3):
                pieces.append(a[:, clr * G + j * g4: clr * G + (j + 1) * g4])
            pieces.append(jnp.zeros((R, Fp - F), a.dtype))
        return jnp.concatenate(pieces, axis=1)

    def body(wi_ref, ws_ref, b1_ref, b2_ref, owi_ref, ows_ref, ob_ref):
        lane = jax.lax.broadcasted_iota(jnp.int32, (1, Op), 1)
        sc = jnp.where(lane < 3 * Fp, 0.5, 1.0)
        owi_ref[...] = (permcols(wi_ref[...]) * sc).astype(jnp.bfloat16)
        pw = (permcols(ws_ref[...]) * sc).astype(jnp.bfloat16)
        ows_ref[...] = jnp.zeros_like(ows_ref)
        for k in range(3):
            ows_ref[k * Fp:k * Fp + F, :] = pw[k * F:(k + 1) * F, :]
        ob_ref[...] = permcols(b1_ref[...] + b2_ref[...]) * sc

    return body


def _make_body(ROWS, Bt, W, C, F, Fp, Op):
    M = Bt * W

    def body(xt_ref, wi_ref, ws_ref, b_ref, out_ref, i2s_ref, h_ref, c_ref):
        @pl.when(pl.program_id(1) == 0)
        def _init():
            h_ref[...] = jnp.zeros_like(h_ref)
            c_ref[...] = jnp.zeros_like(c_ref)

        xblk = xt_ref[...]
        xleft = jnp.concatenate(
            [jnp.zeros((ROWS, Bt, 1, C), xblk.dtype), xblk[:, :, :W - 1, :]],
            axis=2)
        xcat = jnp.concatenate([xleft, xblk], axis=3).reshape(ROWS * M, 2 * C)
        i2s_ref[...] = (
            jnp.dot(xcat, wi_ref[...], preferred_element_type=jnp.float32)
            + b_ref[...]).reshape(ROWS, M, Op)

        ws = ws_ref[...]

        NS = next(n for n in (4, 2, 1) if M % (n * W) == 0)
        Mh = M // NS
        wpos = jax.lax.broadcasted_iota(jnp.int32, (Mh, 1), 0) % W
        not_first = wpos != 0
        not_last = wpos != (W - 1)

        def stream_step(r, s):
            sl = pl.ds(s * Mh, Mh)
            h = h_ref[sl, :]
            zero = jnp.zeros_like(h)
            h_l = jnp.where(not_first, pltpu.roll(h, 1, 0), zero)
            h_r = jnp.where(not_last, pltpu.roll(h, Mh - 1, 0), zero)
            hcat = jnp.concatenate([h_l, h, h_r], axis=1)
            gates = i2s_ref[r, sl, :] + jnp.dot(
                hcat, ws, preferred_element_type=jnp.float32)
            t = jnp.tanh(gates)
            f_g = 0.5 * t[:, 0 * Fp:1 * Fp] + 0.5
            i_g = 0.5 * t[:, 1 * Fp:2 * Fp] + 0.5
            o_g = 0.5 * t[:, 2 * Fp:3 * Fp] + 0.5
            g_g = t[:, 3 * Fp:4 * Fp]
            c_new = f_g * c_ref[sl, :] + i_g * g_g
            c_ref[sl, :] = c_new
            h_new = o_g * jnp.tanh(c_new)
            h_ref[sl, :] = h_new.astype(jnp.bfloat16)
            out_ref[r, sl, :] = h_new[:, :F]

        for r in range(ROWS):
            for s in range(NS):
                stream_step(r, s)

    return body


def kernel(x, w_i2s_masked, b_i2s, w_s2s, b_s2s):
    B, C, H, W = x.shape
    F = w_s2s.shape[1]
    O = 4 * F
    Fp = ((F + 127) // 128) * 128
    Op = 4 * Fp
    BT = 1
    Bt = B // BT
    ROWS = _rows_per_block(H, 12)

    wi3t = jnp.transpose(w_i2s_masked[:, :, 0, 0:2], (2, 1, 0)).reshape(2 * C, O)
    ws3t = jnp.transpose(w_s2s, (2, 1, 0)).reshape(3 * F, O)
    wi, ws, bias = pl.pallas_call(
        _make_prep_body(C, F, Fp, Op),
        out_shape=(jax.ShapeDtypeStruct((2 * C, Op), jnp.bfloat16),
                   jax.ShapeDtypeStruct((3 * Fp, Op), jnp.bfloat16),
                   jax.ShapeDtypeStruct((1, Op), jnp.float32)),
    )(wi3t, ws3t, b_i2s.reshape(1, O), b_s2s.reshape(1, O))

    xt = jnp.transpose(x, (2, 0, 3, 1)).astype(jnp.bfloat16)

    grid = (BT, H // ROWS)
    body = _make_body(ROWS, Bt, W, C, F, Fp, Op)

    out = pl.pallas_call(
        body,
        out_shape=jax.ShapeDtypeStruct((H, B * W, F), jnp.float32),
        grid_spec=pltpu.PrefetchScalarGridSpec(
            num_scalar_prefetch=0,
            grid=grid,
            in_specs=[
                pl.BlockSpec((ROWS, Bt, W, C), lambda bt, rb: (rb, bt, 0, 0)),
                pl.BlockSpec((2 * C, Op), lambda bt, rb: (0, 0)),
                pl.BlockSpec((3 * Fp, Op), lambda bt, rb: (0, 0)),
                pl.BlockSpec((1, Op), lambda bt, rb: (0, 0)),
            ],
            out_specs=pl.BlockSpec((ROWS, Bt * W, F), lambda bt, rb: (rb, bt, 0)),
            scratch_shapes=[
                pltpu.VMEM((ROWS, Bt * W, Op), jnp.float32),
                pltpu.VMEM((Bt * W, Fp), jnp.bfloat16),
                pltpu.VMEM((Bt * W, Fp), jnp.float32),
            ],
        ),
        compiler_params=pltpu.CompilerParams(
            dimension_semantics=("parallel", "arbitrary")),
    )(xt, wi, ws, bias)

    return jnp.transpose(out.reshape(H, B, W, F), (1, 3, 0, 2))


# speedup vs baseline: 1.1081x; 1.0049x over previous
import jax
import jax.numpy as jnp
from jax.experimental import pallas as pl
from jax.experimental.pallas import tpu as pltpu


def _rows_per_block(H, max_rows):
    for r in range(min(H, max_rows), 0, -1):
        if H % r == 0:
            return r
    return 1


def _make_prep_body(C, F, Fp, Op):
    O = 4 * F
    G = O // 3
    g4 = F // 3

    def permcols(a):
        R = a.shape[0]
        pieces = []
        for j in range(4):
            for clr in range(3):
                pieces.append(a[:, clr * G + j * g4: clr * G + (j + 1) * g4])
            pieces.append(jnp.zeros((R, Fp - F), a.dtype))
        return jnp.concatenate(pieces, axis=1)

    def body(wi_ref, ws_ref, b1_ref, b2_ref, owi_ref, ows_ref, ob_ref):
        lane = jax.lax.broadcasted_iota(jnp.int32, (1, Op), 1)
        sc = jnp.where(lane < 3 * Fp, 0.5, 1.0)
        owi_ref[...] = (permcols(wi_ref[...]) * sc).astype(jnp.bfloat16)
        pw = (permcols(ws_ref[...]) * sc).astype(jnp.bfloat16)
        ows_ref[...] = jnp.zeros_like(ows_ref)
        for k in range(3):
            ows_ref[k * Fp:k * Fp + F, :] = pw[k * F:(k + 1) * F, :]
        ob_ref[...] = permcols(b1_ref[...] + b2_ref[...]) * sc

    return body


def _make_body(ROWS, Bt, W, C, F, Fp, Op):
    M = Bt * W

    def body(xt_ref, wi_ref, ws_ref, b_ref, out_ref, i2s_ref, h_ref, c_ref):
        @pl.when(pl.program_id(1) == 0)
        def _init():
            h_ref[...] = jnp.zeros_like(h_ref)
            c_ref[...] = jnp.zeros_like(c_ref)

        xblk = xt_ref[...]
        xleft = jnp.concatenate(
            [jnp.zeros((ROWS, Bt, 1, C), xblk.dtype), xblk[:, :, :W - 1, :]],
            axis=2)
        xcat = jnp.concatenate([xleft, xblk], axis=3).reshape(ROWS * M, 2 * C)
        i2s_ref[...] = (
            jnp.dot(xcat, wi_ref[...], preferred_element_type=jnp.float32)
            + b_ref[...]).reshape(ROWS, M, Op)

        ws = ws_ref[...]

        NS = next(n for n in (4, 2, 1) if M % (n * W) == 0)
        Mh = M // NS
        wpos = jax.lax.broadcasted_iota(jnp.int32, (Mh, 1), 0) % W
        not_first = wpos != 0
        not_last = wpos != (W - 1)

        def stream_step(r, s):
            sl = pl.ds(s * Mh, Mh)
            h = h_ref[sl, :]
            zero = jnp.zeros_like(h)
            h_l = jnp.where(not_first, pltpu.roll(h, 1, 0), zero)
            h_r = jnp.where(not_last, pltpu.roll(h, Mh - 1, 0), zero)
            hcat = jnp.concatenate([h_l, h, h_r], axis=1)
            gates = i2s_ref[r, sl, :] + jnp.dot(
                hcat, ws, preferred_element_type=jnp.float32)
            t = jnp.tanh(gates)
            f_g = 0.5 * t[:, 0 * Fp:1 * Fp] + 0.5
            i_g = 0.5 * t[:, 1 * Fp:2 * Fp] + 0.5
            o_g = 0.5 * t[:, 2 * Fp:3 * Fp] + 0.5
            g_g = t[:, 3 * Fp:4 * Fp]
            c_new = f_g * c_ref[sl, :] + i_g * g_g
            c_ref[sl, :] = c_new
            h_new = o_g * jnp.tanh(c_new)
            h_ref[sl, :] = h_new.astype(jnp.bfloat16)
            out_ref[r, sl, :] = h_new[:, :F]

        for r in range(ROWS):
            for s in range(NS):
                stream_step(r, s)

    return body


def kernel(x, w_i2s_masked, b_i2s, w_s2s, b_s2s):
    B, C, H, W = x.shape
    F = w_s2s.shape[1]
    O = 4 * F
    Fp = ((F + 127) // 128) * 128
    Op = 4 * Fp
    BT = 1
    Bt = B // BT
    ROWS = _rows_per_block(H, 16)

    wi3t = jnp.transpose(w_i2s_masked[:, :, 0, 0:2], (2, 1, 0)).reshape(2 * C, O)
    ws3t = jnp.transpose(w_s2s, (2, 1, 0)).reshape(3 * F, O)
    wi, ws, bias = pl.pallas_call(
        _make_prep_body(C, F, Fp, Op),
        out_shape=(jax.ShapeDtypeStruct((2 * C, Op), jnp.bfloat16),
                   jax.ShapeDtypeStruct((3 * Fp, Op), jnp.bfloat16),
                   jax.ShapeDtypeStruct((1, Op), jnp.float32)),
    )(wi3t, ws3t, b_i2s.reshape(1, O), b_s2s.reshape(1, O))

    xt = jnp.transpose(x, (2, 0, 3, 1)).astype(jnp.bfloat16)

    grid = (BT, H // ROWS)
    body = _make_body(ROWS, Bt, W, C, F, Fp, Op)

    out = pl.pallas_call(
        body,
        out_shape=jax.ShapeDtypeStruct((H, B * W, F), jnp.float32),
        grid_spec=pltpu.PrefetchScalarGridSpec(
            num_scalar_prefetch=0,
            grid=grid,
            in_specs=[
                pl.BlockSpec((ROWS, Bt, W, C), lambda bt, rb: (rb, bt, 0, 0)),
                pl.BlockSpec((2 * C, Op), lambda bt, rb: (0, 0)),
                pl.BlockSpec((3 * Fp, Op), lambda bt, rb: (0, 0)),
                pl.BlockSpec((1, Op), lambda bt, rb: (0, 0)),
            ],
            out_specs=pl.BlockSpec((ROWS, Bt * W, F), lambda bt, rb: (rb, bt, 0)),
            scratch_shapes=[
                pltpu.VMEM((ROWS, Bt * W, Op), jnp.float32),
                pltpu.VMEM((Bt * W, Fp), jnp.bfloat16),
                pltpu.VMEM((Bt * W, Fp), jnp.float32),
            ],
        ),
        compiler_params=pltpu.CompilerParams(
            dimension_semantics=("parallel", "arbitrary")),
    )(xt, wi, ws, bias)

    return jnp.transpose(out.reshape(H, B, W, F), (1, 3, 0, 2))
